# baseline (speedup 1.0000x reference)
import jax
import jax.numpy as jnp
import numpy as np
from jax.experimental import pallas as pl
from jax.experimental.pallas import tpu as pltpu

N_NODES = 21
B = 1024
HALF = B // 2
MAX_RANGE = 10.0
TILE = 128
ROWS = TILE * N_NODES
GRID = HALF // TILE

_PREC = jax.lax.Precision.DEFAULT


def _angle_feat_np():
    bound = np.linspace(-np.pi / 2 - 0.03, np.pi / 2, 21)[:-1]
    angles = bound + np.pi / 20
    return np.stack([np.sin(angles), np.cos(angles)], axis=1).astype(np.float32)


def _feature_map_np():
    S = np.zeros((N_NODES, 24, 7), dtype=np.float32)
    C = np.zeros((N_NODES, 7), dtype=np.float32)
    ang = _angle_feat_np()
    for n in range(20):
        S[n, n, 0] = 1.0 / MAX_RANGE
        C[n, 1] = ang[n, 0]
        C[n, 2] = ang[n, 1]
    for j in range(4):
        S[20, 20 + j, 3 + j] = 1.0
    return S.reshape(N_NODES * 24, 7), C


_S_np, _C_np = _feature_map_np()

_E1_np = (np.arange(256)[None, :] // 64 == np.arange(4)[:, None]).astype(np.float32)
_POOL_np = np.kron(np.full((1, N_NODES), 1.0 / N_NODES, dtype=np.float32),
                   np.eye(TILE, dtype=np.float32))


def _dot(a, b):
    return jnp.dot(a, b, precision=_PREC, preferred_element_type=jnp.float32)


def _leaky(x):
    return jnp.maximum(x, 0.2 * x)


def _elu(x):
    return jnp.maximum(x, jnp.exp(jnp.minimum(x, 0.0)) - 1.0)


def _actor_kernel(sa_ref, sb_ref,
                  s_ref, c_ref, sel_ref, e1_ref,
                  wl1_ref, bl1_ref, wr1_ref, br1_ref, att1_ref, bias1_ref,
                  wl2_ref, bl2_ref, wr2_ref, br2_ref, att2_ref, bias2_ref,
                  pool_ref,
                  w1_ref, b1_ref, w2_ref, b2_ref, w3_ref, b3_ref,
                  out_ref,
                  wn_scr, cn_scr, a1_scr):

    @pl.when(pl.program_id(0) == 0)
    def _fold():
        wl1 = wl1_ref[...]
        wr1 = wr1_ref[...]
        s = s_ref[...]
        c = c_ref[...]
        swl = _dot(s, wl1)
        swr = _dot(s, wr1)
        cwl = _dot(c, wl1) + bl1_ref[...]
        cwr = _dot(c, wr1) + br1_ref[...]
        for n in range(N_NODES):
            swl_n = swl[24 * n:24 * n + 24]
            swr_n = swr[24 * n:24 * n + 24]
            top = slice(48 * n, 48 * n + 24)
            bot = slice(48 * n + 24, 48 * n + 48)
            wn_scr[top, 0:256] = swl_n
            wn_scr[bot, 0:256] = jnp.zeros_like(swl_n)
            wn_scr[top, 256:512] = swl_n
            wn_scr[bot, 256:512] = swr_n
            wn_scr[top, 512:768] = jnp.zeros_like(swl_n)
            wn_scr[bot, 512:768] = swl_n + swr_n
            wn_scr[top, 768:1024] = -swl_n
            wn_scr[bot, 768:1024] = swl_n
        pq_c = cwl + cwr
        cn_scr[:, 0:256] = cwl + bias1_ref[...]
        cn_scr[:, 256:512] = pq_c
        cn_scr[:, 512:768] = pq_c
        cn_scr[:, 768:1024] = jnp.zeros_like(pq_c)
        a1_scr[...] = sel_ref[...] * att1_ref[...]

    sab = jnp.concatenate([sa_ref[...], sb_ref[...]], axis=1)
    cn = cn_scr[...]
    a1 = a1_scr[...]
    e1 = e1_ref[...]

    h1A_parts = []
    h1B_parts = []
    for n in range(N_NODES):
        o = _dot(sab, wn_scr[48 * n:48 * n + 48]) + cn[n:n + 1]
        xlAb = o[:, 0:256]
        p = o[:, 256:512]
        q = o[:, 512:768]
        d = o[:, 768:1024]
        ap = _dot(_leaky(p), a1)
        aq = _dot(_leaky(q), a1)
        m = jnp.maximum(ap, aq)
        wp = 42.0 * jnp.exp(ap - m)
        ws = jnp.exp(aq - m)
        cs = _dot(ws / (wp + ws + 1e-16), e1)
        h1A_parts.append(_elu(xlAb))
        h1B_parts.append(_elu(xlAb + cs * d))
    h1A = jnp.concatenate(h1A_parts, axis=0)
    h1B = jnp.concatenate(h1B_parts, axis=0)

    wl2 = wl2_ref[...]
    bl2 = bl2_ref[...]
    xl2A = _dot(h1A, wl2) + bl2
    xl2B = _dot(h1B, wl2) + bl2
    xr2B = _dot(h1B, wr2_ref[...]) + br2_ref[...]

    att2 = att2_ref[...]
    a_pair2 = jnp.sum(_leaky(xr2B + xl2A) * att2, axis=1, keepdims=True)
    a_self2 = jnp.sum(_leaky(xr2B + xl2B) * att2, axis=1, keepdims=True)
    m2 = jnp.maximum(a_pair2, a_self2)
    wp2 = 42.0 * jnp.exp(a_pair2 - m2)
    ws2 = jnp.exp(a_self2 - m2)
    cs2 = ws2 / (wp2 + ws2 + 1e-16)
    bias2 = bias2_ref[...]
    h2A = xl2A + bias2
    h2B = xl2A + cs2 * (xl2B - xl2A) + bias2

    pool = pool_ref[...]
    gA = _dot(pool, h2A)
    gB = _dot(pool, h2B)

    g = jnp.concatenate([gA, gB], axis=0)
    t = jnp.maximum(_dot(g, w1_ref[...]) + b1_ref[...], 0.0)
    t = jnp.maximum(_dot(t, w2_ref[...]) + b2_ref[...], 0.0)
    o = jnp.tanh(_dot(t, w3_ref[...]) + b3_ref[...])
    out_ref[0] = o[:TILE]
    out_ref[1] = o[TILE:]


def kernel(state24, Wl1, bl1, Wr1, br1, att1, bias1,
           Wl2, bl2, Wr2, br2, att2, bias2,
           W1, b1, W2, b2, W3, b3):
    f32 = jnp.float32
    sa_spec = pl.BlockSpec((TILE, 24), lambda i: (i, 0))
    sb_spec = pl.BlockSpec((TILE, 24), lambda i: (i + GRID, 0))
    full = lambda shape: pl.BlockSpec(shape, lambda i: tuple(0 for _ in shape))
    out_spec = pl.BlockSpec((2, TILE, 2), lambda i: (0, i, 0))

    out = pl.pallas_call(
        _actor_kernel,
        grid=(GRID,),
        in_specs=[
            sa_spec, sb_spec,
            full((N_NODES * 24, 7)), full((N_NODES, 7)),
            full((256, 4)), full((4, 256)),
            full((7, 256)), full((1, 256)), full((7, 256)), full((1, 256)),
            full((256, 1)), full((1, 256)),
            full((256, 64)), full((1, 64)), full((256, 64)), full((1, 64)),
            full((1, 64)), full((1, 64)),
            full((TILE, ROWS)),
            full((64, 256)), full((1, 256)), full((256, 256)), full((1, 256)),
            full((256, 2)), full((1, 2)),
        ],
        out_specs=out_spec,
        out_shape=jax.ShapeDtypeStruct((2, HALF, 2), f32),
        scratch_shapes=[
            pltpu.VMEM((N_NODES * 48, 1024), f32),
            pltpu.VMEM((N_NODES, 1024), f32),
            pltpu.VMEM((256, 4), f32),
        ],
    )(
        state24, state24,
        jnp.asarray(_S_np), jnp.asarray(_C_np),
        jnp.asarray(_E1_np.T), jnp.asarray(_E1_np),
        Wl1, bl1.reshape(1, 256), Wr1, br1.reshape(1, 256),
        att1.reshape(256, 1), bias1.reshape(1, 256),
        Wl2, bl2.reshape(1, 64), Wr2, br2.reshape(1, 64),
        att2.reshape(1, 64), bias2.reshape(1, 64),
        jnp.asarray(_POOL_np),
        W1, b1.reshape(1, 256), W2, b2.reshape(1, 256),
        W3, b3.reshape(1, 2),
    )
    return out.reshape(B, 2)


# speedup vs baseline: 1.3780x; 1.3780x over previous
import jax
import jax.numpy as jnp
import numpy as np
from jax.experimental import pallas as pl
from jax.experimental.pallas import tpu as pltpu

N_NODES = 21
B = 1024
HALF = B // 2
MAX_RANGE = 10.0
TILE = 128
ROWS = TILE * N_NODES
GRID = HALF // TILE

_PREC = jax.lax.Precision.DEFAULT


def _angle_feat_np():
    bound = np.linspace(-np.pi / 2 - 0.03, np.pi / 2, 21)[:-1]
    angles = bound + np.pi / 20
    return np.stack([np.sin(angles), np.cos(angles)], axis=1).astype(np.float32)


def _feature_map_np():
    S = np.zeros((N_NODES, 24, 7), dtype=np.float32)
    C = np.zeros((N_NODES, 7), dtype=np.float32)
    ang = _angle_feat_np()
    for n in range(20):
        S[n, n, 0] = 1.0 / MAX_RANGE
        C[n, 1] = ang[n, 0]
        C[n, 2] = ang[n, 1]
    for j in range(4):
        S[20, 20 + j, 3 + j] = 1.0
    return S.reshape(N_NODES * 24, 7), C


_S_np, _C_np = _feature_map_np()

_E1_np = (np.arange(256)[None, :] // 64 == np.arange(4)[:, None]).astype(np.float32)
_POOL_np = np.kron(np.full((1, N_NODES), 1.0 / N_NODES, dtype=np.float32),
                   np.eye(TILE, dtype=np.float32))


def _dot(a, b):
    return jnp.dot(a, b, precision=_PREC, preferred_element_type=jnp.float32)


def _leaky(x):
    return jnp.maximum(x, 0.2 * x)


def _elu(x):
    return jnp.maximum(x, jnp.exp(jnp.minimum(x, 0.0)) - 1.0)


def _actor_kernel(sa_ref, sb_ref,
                  s_ref, c_ref, sel_ref, e1_ref,
                  wl1_ref, bl1_ref, wr1_ref, br1_ref, att1_ref, bias1_ref,
                  wl2_ref, bl2_ref, wr2_ref, br2_ref, att2_ref, bias2_ref,
                  pool_ref,
                  w1_ref, b1_ref, w2_ref, b2_ref, w3_ref, b3_ref,
                  out_ref,
                  wna_scr, cna_scr, wnb_scr, cnb_scr, a1_scr):

    @pl.when(pl.program_id(0) == 0)
    def _fold():
        wl1 = wl1_ref[...]
        wlr = jnp.concatenate([wl1, wr1_ref[...]], axis=1)
        blr = jnp.concatenate([bl1_ref[...], br1_ref[...]], axis=1)
        s = s_ref[...]
        c = c_ref[...]
        wna_scr[...] = _dot(s, wl1)
        cna_scr[...] = _dot(c, wl1) + bl1_ref[...]
        wnb_scr[...] = _dot(s, wlr)
        cnb_scr[...] = _dot(c, wlr) + blr
        a1_scr[...] = sel_ref[...] * att1_ref[...]

    sa = sa_ref[...]
    sb = sb_ref[...]

    wna = wna_scr[...]
    cna = cna_scr[...]
    wnb = wnb_scr[...]
    cnb = cnb_scr[...]
    xlA = jnp.concatenate(
        [_dot(sa, wna[24 * n:24 * n + 24]) + cna[n:n + 1] for n in range(N_NODES)],
        axis=0)
    xlrB = jnp.concatenate(
        [_dot(sb, wnb[24 * n:24 * n + 24]) + cnb[n:n + 1] for n in range(N_NODES)],
        axis=0)
    xlB = xlrB[:, :256]
    xrB = xlrB[:, 256:]

    a1 = a1_scr[...]
    a_pair = _dot(_leaky(xrB + xlA), a1)
    a_self = _dot(_leaky(xrB + xlB), a1)
    m = jnp.maximum(a_pair, a_self)
    wp = 42.0 * jnp.exp(a_pair - m)
    ws = jnp.exp(a_self - m)
    cs = _dot(ws / (wp + ws + 1e-16), e1_ref[...])
    bias1 = bias1_ref[...]
    h1A = _elu(xlA + bias1)
    h1B = _elu(xlA + cs * (xlB - xlA) + bias1)

    wl2 = wl2_ref[...]
    bl2 = bl2_ref[...]
    xl2A = _dot(h1A, wl2) + bl2
    xl2B = _dot(h1B, wl2) + bl2
    xr2B = _dot(h1B, wr2_ref[...]) + br2_ref[...]

    att2 = att2_ref[...]
    a_pair2 = jnp.sum(_leaky(xr2B + xl2A) * att2, axis=1, keepdims=True)
    a_self2 = jnp.sum(_leaky(xr2B + xl2B) * att2, axis=1, keepdims=True)
    m2 = jnp.maximum(a_pair2, a_self2)
    wp2 = 42.0 * jnp.exp(a_pair2 - m2)
    ws2 = jnp.exp(a_self2 - m2)
    cs2 = ws2 / (wp2 + ws2 + 1e-16)
    bias2 = bias2_ref[...]
    h2A = xl2A + bias2
    h2B = xl2A + cs2 * (xl2B - xl2A) + bias2

    pool = pool_ref[...]
    gA = _dot(pool, h2A)
    gB = _dot(pool, h2B)

    g = jnp.concatenate([gA, gB], axis=0)
    t = jnp.maximum(_dot(g, w1_ref[...]) + b1_ref[...], 0.0)
    t = jnp.maximum(_dot(t, w2_ref[...]) + b2_ref[...], 0.0)
    o = jnp.tanh(_dot(t, w3_ref[...]) + b3_ref[...])
    out_ref[0] = o[:TILE]
    out_ref[1] = o[TILE:]


def kernel(state24, Wl1, bl1, Wr1, br1, att1, bias1,
           Wl2, bl2, Wr2, br2, att2, bias2,
           W1, b1, W2, b2, W3, b3):
    f32 = jnp.float32
    sa_spec = pl.BlockSpec((TILE, 24), lambda i: (i, 0))
    sb_spec = pl.BlockSpec((TILE, 24), lambda i: (i + GRID, 0))
    full = lambda shape: pl.BlockSpec(shape, lambda i: tuple(0 for _ in shape))
    out_spec = pl.BlockSpec((2, TILE, 2), lambda i: (0, i, 0))

    out = pl.pallas_call(
        _actor_kernel,
        grid=(GRID,),
        in_specs=[
            sa_spec, sb_spec,
            full((N_NODES * 24, 7)), full((N_NODES, 7)),
            full((256, 4)), full((4, 256)),
            full((7, 256)), full((1, 256)), full((7, 256)), full((1, 256)),
            full((256, 1)), full((1, 256)),
            full((256, 64)), full((1, 64)), full((256, 64)), full((1, 64)),
            full((1, 64)), full((1, 64)),
            full((TILE, ROWS)),
            full((64, 256)), full((1, 256)), full((256, 256)), full((1, 256)),
            full((256, 2)), full((1, 2)),
        ],
        out_specs=out_spec,
        out_shape=jax.ShapeDtypeStruct((2, HALF, 2), f32),
        scratch_shapes=[
            pltpu.VMEM((N_NODES * 24, 256), f32),
            pltpu.VMEM((N_NODES, 256), f32),
            pltpu.VMEM((N_NODES * 24, 512), f32),
            pltpu.VMEM((N_NODES, 512), f32),
            pltpu.VMEM((256, 4), f32),
        ],
    )(
        state24, state24,
        jnp.asarray(_S_np), jnp.asarray(_C_np),
        jnp.asarray(_E1_np.T), jnp.asarray(_E1_np),
        Wl1, bl1.reshape(1, 256), Wr1, br1.reshape(1, 256),
        att1.reshape(256, 1), bias1.reshape(1, 256),
        Wl2, bl2.reshape(1, 64), Wr2, br2.reshape(1, 64),
        att2.reshape(1, 64), bias2.reshape(1, 64),
        jnp.asarray(_POOL_np),
        W1, b1.reshape(1, 256), W2, b2.reshape(1, 256),
        W3, b3.reshape(1, 2),
    )
    return out.reshape(B, 2)
